# baseline (speedup 1.0000x reference)
import jax
import jax.numpy as jnp
from jax import lax
from jax.experimental import pallas as pl
from jax.experimental.pallas import tpu as pltpu

_B, _L, _T = 16, 512, 64
_W = 72
_D, _K = _T, _T + 1


def _crf_body(emis_ref, tags_ref, tagsn_ref, len_ref, trans_ref, transT_ref,
              start_ref, end_ref, out_ref, ee_ref):
    emis = emis_ref[...]
    tags3 = tags_ref[...][:, :, None]
    tagsn3 = tagsn_ref[...][:, :, None]
    lens = len_ref[...]
    lens3 = lens.reshape(1, _B, 1)

    iota_j = lax.broadcasted_iota(jnp.int32, (_L, _B, _T), 2)
    tpos3 = lax.broadcasted_iota(jnp.int32, (_L, _B, _T), 0)

    oh = (iota_j == tags3).astype(jnp.float32)
    valid = tpos3 < lens3
    emit_sum = jnp.sum(jnp.sum(jnp.where(valid, emis * oh, 0.0), axis=2),
                       axis=0)

    rows = lax.dot_general(oh.reshape(_L * _B, _T), trans_ref[...],
                           (((1,), (0,)), ((), ())),
                           preferred_element_type=jnp.float32)
    rows = rows.reshape(_L, _B, _T)
    ohn = (iota_j == tagsn3).astype(jnp.float32)
    validn = (tpos3 + 1) < lens3
    trans_sum = jnp.sum(jnp.sum(jnp.where(validn, rows * ohn, 0.0), axis=2),
                        axis=0)

    start_sc = jnp.sum(start_ref[...] * oh[0], axis=1)
    lastmask = ((tpos3[:, :, 0] + 1) == lens3[:, :, 0]).astype(jnp.int32)
    last_tag = jnp.sum(lastmask * tags_ref[...], axis=0)
    iota_bt = lax.broadcasted_iota(jnp.int32, (_B, _T), 1)
    end_oh = (iota_bt == last_tag[:, None]).astype(jnp.float32)
    end_sc = jnp.sum(end_ref[...] * end_oh, axis=1)
    log_s = (start_sc + emit_sum + trans_sum + end_sc)[:, None]

    live = jnp.where(valid, jnp.exp(emis), 0.0)
    iota_r = lax.broadcasted_iota(jnp.int32, (_L, _B, _W - _T), 2)
    dump = (tpos3[:, :, :1] == lens3).astype(jnp.float32)
    right = jnp.where(iota_r == 0, dump,
                      jnp.where(iota_r == 1, 1.0, 0.0))
    ee_ref[...] = jnp.concatenate([live, right],
                                  axis=2).astype(jnp.bfloat16)

    e_end = jnp.exp(end_ref[...])
    e_end_col = jnp.transpose(e_end, (1, 0))
    ic = lax.broadcasted_iota(jnp.int32, (_T, _W - _T), 1)
    top = jnp.concatenate(
        [jnp.exp(trans_ref[...]),
         jnp.where(ic == 0, e_end_col, 0.0)], axis=1)
    ir2 = lax.broadcasted_iota(jnp.int32, (_W - _T, _W), 0)
    ic2 = lax.broadcasted_iota(jnp.int32, (_W - _T, _W), 1)
    bottom = ((ir2 <= 1) & (ic2 == _K)).astype(jnp.float32)
    E = jnp.concatenate([top, bottom], axis=0).astype(jnp.bfloat16)

    topT = jnp.concatenate(
        [jnp.exp(transT_ref[...]), jnp.zeros((_T, _W - _T), jnp.float32)],
        axis=1)
    e_end_pad = jnp.concatenate(
        [e_end, jnp.zeros((1, _W - _T), jnp.float32)], axis=1)
    botT = jnp.where(ir2 == 0, jnp.broadcast_to(e_end_pad, (_W - _T, _W)),
                     jnp.where((ir2 == 1) & ((ic2 == _D) | (ic2 == _K)),
                               1.0, 0.0))
    ET = jnp.concatenate([topT, botT], axis=0).astype(jnp.bfloat16)

    iota_w = lax.broadcasted_iota(jnp.int32, (_B, _W), 1)
    u0 = jnp.concatenate(
        [jnp.exp(start_ref[...]) * jnp.exp(emis[0]),
         jnp.zeros((_B, _W - _T), jnp.float32)],
        axis=1).astype(jnp.bfloat16)
    w0 = ((iota_w == _D) | (iota_w == _K)).astype(jnp.bfloat16)

    def step_f(t, u):
        s = lax.dot_general(u, E, (((1,), (0,)), ((), ())),
                            preferred_element_type=jnp.float32)
        return s.astype(jnp.bfloat16) * ee_ref[t]

    def step_b(t, w):
        h = w * ee_ref[t]
        s = lax.dot_general(h, ET, (((1,), (0,)), ((), ())),
                            preferred_element_type=jnp.float32)
        return s.astype(jnp.bfloat16)

    def rescale(a, c):
        m = jnp.max(a.astype(jnp.float32), axis=1, keepdims=True)
        return (a.astype(jnp.float32) / m).astype(jnp.bfloat16), c + jnp.log(m)

    u, w = u0, w0
    for i in range(1, 8):
        u = step_f(i, u)
    for i in range(8):
        w = step_b(511 - i, w)
    zero_c = jnp.zeros((_B, 1), jnp.float32)
    u, cf = rescale(u, zero_c)
    w, cb = rescale(w, zero_c)

    def block(i, carry):
        u, w, cf, cb = carry
        for q in range(8):
            u = step_f(8 + 8 * i + q, u)
            w = step_b(503 - 8 * i - q, w)
        u, cf = rescale(u, cf)
        w, cb = rescale(w, cb)
        return (u, w, cf, cb)

    u, w, cf, cb = lax.fori_loop(0, 31, block, (u, w, cf, cb))

    z = jnp.sum(u.astype(jnp.float32) * w.astype(jnp.float32),
                axis=1, keepdims=True)
    log_z = cf + cb + jnp.log(z)

    out_ref[...] = log_s - log_z


def kernel(emissions, tags, lengths, transitions, start_transitions,
           end_transitions):
    emis_t = jnp.transpose(emissions, (1, 0, 2))
    tags_t = jnp.transpose(tags, (1, 0))
    tagsn_t = jnp.concatenate(
        [tags_t[1:], jnp.zeros((1, _B), jnp.int32)], axis=0)
    lens = jnp.maximum(lengths, 1).astype(jnp.int32)[:, None]
    out = pl.pallas_call(
        _crf_body,
        out_shape=jax.ShapeDtypeStruct((_B, 1), jnp.float32),
        scratch_shapes=[pltpu.VMEM((_L, _B, _W), jnp.bfloat16)],
    )(emis_t, tags_t, tagsn_t, lens, transitions,
      jnp.transpose(transitions, (1, 0)),
      start_transitions[None, :], end_transitions[None, :])
    return out[:, 0]
